# Initial kernel scaffold; baseline (speedup 1.0000x reference)
import jax
import jax.numpy as jnp
from jax.experimental import pallas as pl


def kernel(atomic_number, W):
    raise NotImplementedError("write your pallas kernel here")


# speedup vs baseline: 71.8639x; 71.8639x over previous
import functools

import jax
import jax.numpy as jnp
from jax import lax
from jax.experimental import pallas as pl
from jax.experimental.pallas import tpu as pltpu
from jax.experimental.pallas import tpu_sc as plsc

_B = 16384
_N = 512
_MAX_ELEM = 94
_NC = 2
_NS = 16
_NW = _NC * _NS
_ROWS_PER_W = _B // _NW
_CHUNK = 64
_NCHUNK = _ROWS_PER_W // _CHUNK
_GROUPS = _CHUNK // 16
_TBL = 256 * _MAX_ELEM


def _body(idx_hbm, tbl_hbm, out_hbm, idx_v, tbl_v, out_v):
    wid = lax.axis_index("s") * _NC + lax.axis_index("c")
    row0 = wid * _ROWS_PER_W
    pltpu.sync_copy(tbl_hbm, tbl_v)
    lane = lax.iota(jnp.int32, 16)

    def chunk_body(c, carry):
        pltpu.sync_copy(
            idx_hbm.at[pl.ds((row0 + c * _CHUNK) * _N, _CHUNK * _N)], idx_v)

        def group_body(g, carry):
            rowbase = (g * 16 + lane) * _N

            def j_body(jj, acc):
                pos = rowbase + jj * 2
                v0 = plsc.load_gather(idx_v, [pos])
                v1 = plsc.load_gather(idx_v, [pos + 1])
                comb = v0 + (v1 << 8)
                return acc + plsc.load_gather(tbl_v, [comb])

            acc = lax.fori_loop(0, _N // 2, j_body, jnp.zeros((16,), jnp.float32))
            out_v[pl.ds(c * _CHUNK + g * 16, 16)] = acc * jnp.float32(1.0 / _N)
            return carry

        return lax.fori_loop(0, _GROUPS, group_body, carry)

    lax.fori_loop(0, _NCHUNK, chunk_body, jnp.int32(0))
    pltpu.sync_copy(out_v, out_hbm.at[pl.ds(row0, _ROWS_PER_W)])


@jax.jit
def kernel(atomic_number, W):
    w = W.reshape(-1).astype(jnp.float32)
    wpad = jnp.zeros((256,), jnp.float32).at[:_MAX_ELEM].set(w)
    tbl = (w[:, None] + wpad[None, :]).reshape(-1)

    mesh = plsc.VectorSubcoreMesh(core_axis_name="c", subcore_axis_name="s")
    run = functools.partial(
        pl.kernel,
        mesh=mesh,
        out_type=jax.ShapeDtypeStruct((_B,), jnp.float32),
        scratch_types=[
            pltpu.VMEM((_CHUNK * _N,), jnp.int32),
            pltpu.VMEM((_TBL,), jnp.float32),
            pltpu.VMEM((_ROWS_PER_W,), jnp.float32),
        ],
        compiler_params=pltpu.CompilerParams(
            needs_layout_passes=False, use_tc_tiling_on_sc=False),
    )(_body)
    return run(atomic_number.reshape(-1), tbl)


# speedup vs baseline: 77.0432x; 1.0721x over previous
import functools

import jax
import jax.numpy as jnp
from jax import lax
from jax.experimental import pallas as pl
from jax.experimental.pallas import tpu as pltpu
from jax.experimental.pallas import tpu_sc as plsc

_B = 16384
_N = 512
_MAX_ELEM = 94
_NC = 2
_NS = 16
_NW = _NC * _NS
_ROWS_PER_W = _B // _NW
_CHUNK = 64
_NCHUNK = _ROWS_PER_W // _CHUNK
_GROUPS = _CHUNK // 16
_TBL = 256 * _MAX_ELEM


def _body(idx_hbm, tbl_hbm, out_hbm, idx_a, idx_b, tbl_v, out_v, sem_a, sem_b):
    wid = lax.axis_index("s") * _NC + lax.axis_index("c")
    row0 = wid * _ROWS_PER_W
    pltpu.sync_copy(tbl_hbm, tbl_v)
    lane = lax.iota(jnp.int32, 16)

    bufs = [idx_a, idx_b]
    sems = [sem_a, sem_b]

    def start(c):
        pltpu.async_copy(
            idx_hbm.at[pl.ds((row0 + c * _CHUNK) * _N, _CHUNK * _N)],
            bufs[c % 2], sems[c % 2])

    start(0)
    for c in range(_NCHUNK):
        pltpu.make_async_copy(
            idx_hbm.at[pl.ds((row0 + c * _CHUNK) * _N, _CHUNK * _N)],
            bufs[c % 2], sems[c % 2]).wait()
        if c + 1 < _NCHUNK:
            start(c + 1)
        buf = bufs[c % 2]

        @plsc.parallel_loop(0, _GROUPS)
        def _group(g):
            rowbase = (g * 16 + lane) * _N

            @plsc.parallel_loop(0, _N // 4, unroll=4,
                                carry=(jnp.zeros((16,), jnp.float32),
                                       jnp.zeros((16,), jnp.float32)))
            def _accs(jj, accs):
                a0, a1 = accs
                pos = rowbase + jj * 4
                v0 = plsc.load_gather(buf, [pos])
                v1 = plsc.load_gather(buf, [pos + 1])
                v2 = plsc.load_gather(buf, [pos + 2])
                v3 = plsc.load_gather(buf, [pos + 3])
                a0 = a0 + plsc.load_gather(tbl_v, [v0 + (v1 << 8)])
                a1 = a1 + plsc.load_gather(tbl_v, [v2 + (v3 << 8)])
                return a0, a1

            a0, a1 = _accs
            out_v[pl.ds(c * _CHUNK + g * 16, 16)] = (a0 + a1) * jnp.float32(1.0 / _N)

    pltpu.sync_copy(out_v, out_hbm.at[pl.ds(row0, _ROWS_PER_W)])


@jax.jit
def kernel(atomic_number, W):
    w = W.reshape(-1).astype(jnp.float32)
    wpad = jnp.zeros((256,), jnp.float32).at[:_MAX_ELEM].set(w)
    tbl = (w[:, None] + wpad[None, :]).reshape(-1)

    mesh = plsc.VectorSubcoreMesh(core_axis_name="c", subcore_axis_name="s")
    run = functools.partial(
        pl.kernel,
        mesh=mesh,
        out_type=jax.ShapeDtypeStruct((_B,), jnp.float32),
        scratch_types=[
            pltpu.VMEM((_CHUNK * _N,), jnp.int32),
            pltpu.VMEM((_CHUNK * _N,), jnp.int32),
            pltpu.VMEM((_TBL,), jnp.float32),
            pltpu.VMEM((_ROWS_PER_W,), jnp.float32),
            pltpu.SemaphoreType.DMA,
            pltpu.SemaphoreType.DMA,
        ],
        compiler_params=pltpu.CompilerParams(
            needs_layout_passes=False, use_tc_tiling_on_sc=False),
    )(_body)
    return run(atomic_number.reshape(-1), tbl)


# speedup vs baseline: 151.6595x; 1.9685x over previous
import functools

import jax
import jax.numpy as jnp
from jax import lax
from jax.experimental import pallas as pl
from jax.experimental.pallas import tpu as pltpu
from jax.experimental.pallas import tpu_sc as plsc

_B = 16384
_N = 512
_MAX_ELEM = 94
_NC = 2
_NS = 16
_NW = _NC * _NS
_ROWS_PER_W = _B // _NW
_CHUNK = 64
_NCHUNK = _ROWS_PER_W // _CHUNK
_GROUPS = _CHUNK // 16
_TBL = 256 * _MAX_ELEM


def _body(idx_hbm, tbl_hbm, out_hbm, idx_a, idx_b, tbl_v, out_v, sem_a, sem_b):
    wid = lax.axis_index("s") * _NC + lax.axis_index("c")
    row0 = wid * _ROWS_PER_W
    pltpu.sync_copy(tbl_hbm, tbl_v)
    lane = lax.iota(jnp.int32, 16)
    zero = jnp.zeros((16,), jnp.float32)

    bufs = [idx_a, idx_b]
    sems = [sem_a, sem_b]

    def start(c):
        pltpu.async_copy(
            idx_hbm.at[pl.ds(row0 + c * _CHUNK, _CHUNK)],
            bufs[c % 2], sems[c % 2])

    start(0)
    for c in range(_NCHUNK):
        pltpu.make_async_copy(
            idx_hbm.at[pl.ds(row0 + c * _CHUNK, _CHUNK)],
            bufs[c % 2], sems[c % 2]).wait()
        if c + 1 < _NCHUNK:
            start(c + 1)
        buf = bufs[c % 2]

        @plsc.parallel_loop(0, _GROUPS)
        def _group(g):

            def row_body(rr, res):
                r = g * 16 + rr

                @plsc.parallel_loop(0, _N // 64, unroll=4,
                                    carry=(zero, zero))
                def _accs(jj, accs):
                    a0, a1 = accs
                    base = jj * 64
                    v0 = buf[r, pl.ds(base, 16)]
                    v1 = buf[r, pl.ds(base + 16, 16)]
                    v2 = buf[r, pl.ds(base + 32, 16)]
                    v3 = buf[r, pl.ds(base + 48, 16)]
                    a0 = a0 + plsc.load_gather(tbl_v, [v0 + (v1 << 8)])
                    a1 = a1 + plsc.load_gather(tbl_v, [v2 + (v3 << 8)])
                    return a0, a1

                a0, a1 = _accs
                tot = jnp.sum(a0 + a1)
                return jnp.where(lane == rr, tot, res)

            res = lax.fori_loop(0, 16, row_body, zero)
            out_v[pl.ds(c * _CHUNK + g * 16, 16)] = res * jnp.float32(1.0 / _N)

    pltpu.sync_copy(out_v, out_hbm.at[pl.ds(row0, _ROWS_PER_W)])


@jax.jit
def kernel(atomic_number, W):
    w = W.reshape(-1).astype(jnp.float32)
    wpad = jnp.zeros((256,), jnp.float32).at[:_MAX_ELEM].set(w)
    tbl = (w[:, None] + wpad[None, :]).reshape(-1)

    mesh = plsc.VectorSubcoreMesh(core_axis_name="c", subcore_axis_name="s")
    run = functools.partial(
        pl.kernel,
        mesh=mesh,
        out_type=jax.ShapeDtypeStruct((_B,), jnp.float32),
        scratch_types=[
            pltpu.VMEM((_CHUNK, _N), jnp.int32),
            pltpu.VMEM((_CHUNK, _N), jnp.int32),
            pltpu.VMEM((_TBL,), jnp.float32),
            pltpu.VMEM((_ROWS_PER_W,), jnp.float32),
            pltpu.SemaphoreType.DMA,
            pltpu.SemaphoreType.DMA,
        ],
        compiler_params=pltpu.CompilerParams(
            needs_layout_passes=False, use_tc_tiling_on_sc=False),
    )(_body)
    return run(atomic_number, tbl)


# speedup vs baseline: 234.0302x; 1.5431x over previous
import functools

import jax
import jax.numpy as jnp
from jax import lax
from jax.experimental import pallas as pl
from jax.experimental.pallas import tpu as pltpu
from jax.experimental.pallas import tpu_sc as plsc

_B = 16384
_N = 512
_MAX_ELEM = 94
_NC = 2
_NS = 16
_NW = _NC * _NS
_ROWS_PER_W = _B // _NW
_CHUNK = 64
_NCHUNK = _ROWS_PER_W // _CHUNK
_GROUPS = _CHUNK // 16
_TBL = 256 * _MAX_ELEM


def _body(idx_hbm, tbl_hbm, out_hbm, idx_a, idx_b, tbl_v, out_v, sem_a, sem_b):
    wid = lax.axis_index("s") * _NC + lax.axis_index("c")
    row0 = wid * _ROWS_PER_W
    pltpu.sync_copy(tbl_hbm, tbl_v)
    lane = lax.iota(jnp.int32, 16)
    zero = jnp.zeros((16,), jnp.float32)

    bufs = [idx_a, idx_b]
    sems = [sem_a, sem_b]

    def start(c):
        pltpu.async_copy(
            idx_hbm.at[pl.ds(row0 + c * _CHUNK, _CHUNK)],
            bufs[c % 2], sems[c % 2])

    start(0)
    for c in range(_NCHUNK):
        pltpu.make_async_copy(
            idx_hbm.at[pl.ds(row0 + c * _CHUNK, _CHUNK)],
            bufs[c % 2], sems[c % 2]).wait()
        if c + 1 < _NCHUNK:
            start(c + 1)
        buf = bufs[c % 2]

        @plsc.parallel_loop(0, _GROUPS)
        def _group(g):

            def row_body(rr, res):
                r = g * 16 + rr

                @plsc.parallel_loop(0, _N // 64, unroll=4,
                                    carry=(zero, zero))
                def _accs(jj, accs):
                    a0, a1 = accs
                    base = jj * 64
                    v0 = buf[r, pl.ds(base, 16)]
                    v1 = buf[r, pl.ds(base + 16, 16)]
                    v2 = buf[r, pl.ds(base + 32, 16)]
                    v3 = buf[r, pl.ds(base + 48, 16)]
                    a0 = a0 + plsc.load_gather(tbl_v, [v0 + (v1 << 8)])
                    a1 = a1 + plsc.load_gather(tbl_v, [v2 + (v3 << 8)])
                    return a0, a1

                a0, a1 = _accs
                tot = jnp.sum(a0 + a1)
                return jnp.where(lane == rr, tot, res)

            res = lax.fori_loop(0, 16, row_body, zero)
            out_v[pl.ds(c * _CHUNK + g * 16, 16)] = res * jnp.float32(1.0 / _N)

    pltpu.sync_copy(out_v, out_hbm.at[pl.ds(row0, _ROWS_PER_W)])


@jax.jit
def kernel(atomic_number, W):
    w = W.reshape(-1).astype(jnp.float32)
    wpad = jnp.zeros((256,), jnp.float32).at[:_MAX_ELEM].set(w)
    tbl = (w[:, None] + wpad[None, :]).reshape(-1)

    mesh = plsc.VectorSubcoreMesh(core_axis_name="c", subcore_axis_name="s")
    run = functools.partial(
        pl.kernel,
        mesh=mesh,
        out_type=jax.ShapeDtypeStruct((_B,), jnp.float32),
        scratch_types=[
            pltpu.VMEM((_CHUNK, _N), jnp.int32),
            pltpu.VMEM((_CHUNK, _N), jnp.int32),
            pltpu.VMEM((_TBL,), jnp.float32),
            pltpu.VMEM((_ROWS_PER_W,), jnp.float32),
            pltpu.SemaphoreType.DMA,
            pltpu.SemaphoreType.DMA,
        ],
        compiler_params=pltpu.CompilerParams(
            needs_layout_passes=False, use_tc_tiling_on_sc=True),
    )(_body)
    return run(atomic_number, tbl)
